# baseline (speedup 1.0000x reference)
import jax
import jax.numpy as jnp
from jax import lax
from jax.experimental import pallas as pl
from jax.experimental.pallas import tpu as pltpu
from jax.experimental.pallas import tpu_sc as plsc

_N_FRAG = 320000
_SEG_REAL = 500000
_CELL_N = 500
_GENE_N = 1000
_D = 16
_PK = 128 // _D

_R_BLK = 4096
_NPAD = 79 * _R_BLK

_NC, _NS = 2, 16
_R_SEG = 125952
_N_UNIT = 4
_S_PAD = _R_SEG * _N_UNIT
_DUMP = _R_SEG
_SPM_ROWS = _R_SEG + 16
_ZSTRIPE = _SPM_ROWS // _NS
_ZTAIL = _ZSTRIPE - 61 * 128
_DSTRIPE = _R_SEG // _NS
_CHUNK = 128

_RO_BLK = 4096


def _sin_cos(x):
    f32 = jnp.float32
    q = jnp.round(x * f32(0.6366197723675814))
    i = q.astype(jnp.int32)
    r = ((x - q * f32(1.5703125))
         - q * f32(4.837512969970703125e-4)) - q * f32(7.549789948768648e-8)
    z = r * r
    sp = r + r * z * (f32(-1.6666654611e-1)
                      + z * (f32(8.3321608736e-3) + z * f32(-1.9515295891e-4)))
    cp = (f32(1.0) - f32(0.5) * z
          + z * z * (f32(4.166664568298827e-2)
                     + z * (f32(-1.388731625493765e-3)
                            + z * f32(2.443315711809948e-5))))
    even = (i & 1) == 0
    s_out = jnp.where(even, sp, cp)
    s_out = jnp.where((i & 2) != 0, -s_out, s_out)
    c_out = jnp.where(even, cp, sp)
    c_out = jnp.where(((i + 1) & 2) != 0, -c_out, c_out)
    return s_out, c_out


def _embed_body(c_ref, fab_ref, ws_ref, wc_ref, b1_ref, w2b_ref, b2b_ref,
                h_ref):
    c0 = c_ref[:, 0:1]
    c1 = c_ref[:, 1:2]
    x = c0 * fab_ref[0:1, :] + c1 * fab_ref[1:2, :]
    s, co = _sin_cos(x)
    z = (jnp.dot(s, ws_ref[...], preferred_element_type=jnp.float32)
         + jnp.dot(co, wc_ref[...], preferred_element_type=jnp.float32)
         + b1_ref[...])
    h1 = jax.nn.sigmoid(z)
    h_ref[...] = (jnp.dot(h1, w2b_ref[...], preferred_element_type=jnp.float32)
                  + b2b_ref[...])


def _embed(coords_pad, fab, ws, wc, b1p, w2big, b2big):
    full = lambda shape: pl.BlockSpec(shape, lambda i: tuple(0 for _ in shape))
    return pl.pallas_call(
        _embed_body,
        grid=(_NPAD // _R_BLK,),
        in_specs=[
            pl.BlockSpec((_R_BLK, 2), lambda i: (i, 0)),
            full((2, 128)),
            full((128, _D)), full((128, _D)), full((1, _D)),
            full((_D, _D)), full((1, _D)),
        ],
        out_specs=pl.BlockSpec((_R_BLK, _D), lambda i: (i, 0)),
        out_shape=jax.ShapeDtypeStruct((_NPAD, _D), jnp.float32),
    )(coords_pad, fab, ws, wc, b1p, w2big, b2big)


def _pool_body(h_hbm, ids_hbm, bnd_hbm, pooled_hbm,
               acc, hv2, idsv2, idxv, bnd_v, hsem, isem):
    c = lax.axis_index("c")
    s = lax.axis_index("s")

    pltpu.sync_copy(bnd_hbm, bnd_v)
    bv = bnd_v[...]

    for p in range(2):
        u = c * 2 + p
        r0 = u * _R_SEG
        f0 = jnp.where(c == 0, bv[p], bv[2 + p])
        f1 = jnp.where(c == 0, bv[p + 1], bv[3 + p])
        a0 = (f0 // 8) * 8
        nblk = (f1 - a0 + (_CHUNK - 1)) // _CHUNK
        nmy = jnp.maximum((nblk - s + (_NS - 1)) // _NS, 0)

        for j in range(_CHUNK):
            hv2[0, j, :] = jnp.zeros((16,), jnp.float32)
        zoff = s * _ZSTRIPE
        for k in range(61):
            pltpu.sync_copy(hv2.at[0], acc.at[pl.ds(zoff + k * _CHUNK, _CHUNK)])
        pltpu.sync_copy(hv2.at[0, pl.ds(0, _ZTAIL)],
                        acc.at[pl.ds(zoff + 61 * _CHUNK, _ZTAIL)])
        plsc.subcore_barrier()

        def start(j, b):
            base = a0 + (s + j * _NS) * _CHUNK
            pltpu.async_copy(ids_hbm.at[pl.ds(base, _CHUNK)], idsv2.at[b],
                             isem.at[b])
            pltpu.async_copy(h_hbm.at[pl.ds(base, _CHUNK)], hv2.at[b],
                             hsem.at[b])

        def wait(b):
            pltpu.make_async_copy(ids_hbm.at[pl.ds(0, _CHUNK)], idsv2.at[b],
                                  isem.at[b]).wait()
            pltpu.make_async_copy(h_hbm.at[pl.ds(0, _CHUNK)], hv2.at[b],
                                  hsem.at[b]).wait()

        @pl.when(nmy > 0)
        def _():
            start(0, 0)

        def outer(k, carry):
            for b in range(2):
                j = 2 * k + b

                @pl.when(j < nmy)
                def _():
                    wait(b)

                    @pl.when(j + 1 < nmy)
                    def _():
                        start(j + 1, 1 - b)
                    for t in range(_CHUNK // 16):
                        iv = idsv2[b, pl.ds(t * 16, 16)]
                        ok = (iv >= r0) & (iv < r0 + _R_SEG)
                        idxv[pl.ds(t * 16, 16)] = jnp.where(ok, iv - r0, _DUMP)
                    pltpu.sync_copy(hv2.at[b], acc.at[idxv], add=True)
            return carry

        lax.fori_loop(0, (nmy + 1) // 2, outer, 0)
        plsc.subcore_barrier()

        doff = s * _DSTRIPE
        pltpu.sync_copy(acc.at[pl.ds(doff, _DSTRIPE)],
                        pooled_hbm.at[pl.ds(r0 + doff, _DSTRIPE)])
        plsc.subcore_barrier()


def _pool(h, ids_pad, bnd16):
    call = pl.kernel(
        _pool_body,
        out_type=jax.ShapeDtypeStruct((_S_PAD, _D), jnp.float32),
        mesh=plsc.VectorSubcoreMesh(core_axis_name="c", subcore_axis_name="s",
                                    num_cores=_NC, num_subcores=_NS),
        scratch_types=[
            pltpu.VMEM_SHARED((_SPM_ROWS, _D), jnp.float32),
            pltpu.VMEM((2, _CHUNK, _D), jnp.float32),
            pltpu.VMEM((2, _CHUNK), jnp.int32),
            pltpu.VMEM((_CHUNK,), jnp.int32),
            pltpu.VMEM((16,), jnp.int32),
            pltpu.SemaphoreType.DMA((2,)),
            pltpu.SemaphoreType.DMA((2,)),
        ],
        compiler_params=pltpu.CompilerParams(use_tc_tiling_on_sc=False),
    )
    return call(h, ids_pad, bnd16)


def _readout_body(p_ref, w3t_ref, b3_ref, w4c_ref, b4t_ref, o_ref):
    z = jax.nn.sigmoid(
        jnp.dot(p_ref[...], w3t_ref[...], preferred_element_type=jnp.float32)
        + b3_ref[...])
    o = (jnp.dot(z, w4c_ref[...], preferred_element_type=jnp.float32)
         + b4t_ref[...])
    o_ref[...] = o[:, 0:1]


def _readout(pooled, w3t, b3p, w4c, b4t):
    full = lambda shape: pl.BlockSpec(shape, lambda i: tuple(0 for _ in shape))
    return pl.pallas_call(
        _readout_body,
        grid=(_S_PAD // _RO_BLK,),
        in_specs=[
            pl.BlockSpec((_RO_BLK, _D), lambda i: (i, 0)),
            full((_D, _D)), full((1, _D)), full((_D, _PK)),
            full((1, _PK)),
        ],
        out_specs=pl.BlockSpec((_RO_BLK, 1), lambda i: (i, 0)),
        out_shape=jax.ShapeDtypeStruct((_S_PAD, 1), jnp.float32),
    )(pooled, w3t, b3p, w4c, b4t)


def kernel(coordinates, local_cellxgene_ix, cell_n, gene_n, frequencies,
           shifts, W1, b1, W2, b2, W3, b3, W4, b4):
    f32 = jnp.float32
    n = coordinates.shape[0]
    ids = local_cellxgene_ix.astype(jnp.int32)

    coords_pad = jnp.zeros((_NPAD, 2), f32).at[:n].set(coordinates)
    ids_pad = jnp.full((_NPAD,), 1 << 30, jnp.int32).at[:n].set(ids)

    funiq = frequencies[0:100:2]
    fab = jnp.zeros((2, 128), f32)
    fab = fab.at[0, 0:50].set(funiq).at[1, 64:114].set(funiq)
    ws = jnp.zeros((128, _D), f32)
    ws = ws.at[0:50, 0:10].set(W1[:, 0:100:2].T)
    ws = ws.at[64:114, 0:10].set(W1[:, 100:200:2].T)
    wc = jnp.zeros((128, _D), f32)
    wc = wc.at[0:50, 0:10].set(W1[:, 1:100:2].T)
    wc = wc.at[64:114, 0:10].set(W1[:, 101:200:2].T)
    b1p = jnp.zeros((1, _D), f32).at[0, :10].set(b1)
    w2big = jnp.zeros((_D, _D), f32).at[:10, :10].set(W2.T)
    b2big = jnp.zeros((1, _D), f32).at[0, :10].set(b2)

    h = _embed(coords_pad, fab, ws, wc, b1p, w2big, b2big)

    bnds = jnp.searchsorted(
        ids, jnp.arange(0, _S_PAD + 1, _R_SEG, dtype=jnp.int32),
        side="left").astype(jnp.int32)
    bnd16 = jnp.zeros((16,), jnp.int32).at[:5].set(bnds)

    pooled = _pool(h, ids_pad, bnd16)

    w3t = jnp.zeros((_D, _D), f32).at[:10, :10].set(W3.T)
    b3p = jnp.zeros((1, _D), f32).at[0, :10].set(b3)
    w4c = jnp.zeros((_D, _PK), f32).at[:10, 0].set(W4[0])
    b4t = jnp.full((1, _PK), b4[0], f32)

    out = _readout(pooled, w3t, b3p, w4c, b4t)
    return out.reshape(_S_PAD)[:_SEG_REAL].reshape(_CELL_N, _GENE_N)


# speedup vs baseline: 1.3732x; 1.3732x over previous
import jax
import jax.numpy as jnp
from jax import lax
from jax.experimental import pallas as pl
from jax.experimental.pallas import tpu as pltpu
from jax.experimental.pallas import tpu_sc as plsc

_N_FRAG = 320000
_SEG_REAL = 500000
_CELL_N = 500
_GENE_N = 1000
_D = 16
_PK = 128 // _D

_R_BLK = 4096
_NPAD = 79 * _R_BLK

_NC, _NS = 2, 16
_R_SEG = 125952
_N_UNIT = 4
_S_PAD = _R_SEG * _N_UNIT
_DUMP = _R_SEG
_SPM_ROWS = _R_SEG + 16
_ZSTRIPE = _SPM_ROWS // _NS
_ZTAIL = _ZSTRIPE - 61 * 128
_DSTRIPE = _R_SEG // _NS
_CHUNK = 128

_RO_BLK = 512


def _sin_cos(x):
    f32 = jnp.float32
    q = jnp.round(x * f32(0.6366197723675814))
    i = q.astype(jnp.int32)
    r = ((x - q * f32(1.5703125))
         - q * f32(4.837512969970703125e-4)) - q * f32(7.549789948768648e-8)
    z = r * r
    sp = r + r * z * (f32(-1.6666654611e-1)
                      + z * (f32(8.3321608736e-3) + z * f32(-1.9515295891e-4)))
    cp = (f32(1.0) - f32(0.5) * z
          + z * z * (f32(4.166664568298827e-2)
                     + z * (f32(-1.388731625493765e-3)
                            + z * f32(2.443315711809948e-5))))
    even = (i & 1) == 0
    s_out = jnp.where(even, sp, cp)
    s_out = jnp.where((i & 2) != 0, -s_out, s_out)
    c_out = jnp.where(even, cp, sp)
    c_out = jnp.where(((i + 1) & 2) != 0, -c_out, c_out)
    return s_out, c_out


def _embed_body(c_ref, fab_ref, ws_ref, wc_ref, b1_ref, w2b_ref, b2b_ref,
                h_ref):
    c0 = c_ref[:, 0:1]
    c1 = c_ref[:, 1:2]
    x = c0 * fab_ref[0:1, :] + c1 * fab_ref[1:2, :]
    s, co = _sin_cos(x)
    z = (jnp.dot(s, ws_ref[...], preferred_element_type=jnp.float32)
         + jnp.dot(co, wc_ref[...], preferred_element_type=jnp.float32)
         + b1_ref[...])
    h1 = jax.nn.sigmoid(z)
    h_ref[...] = (jnp.dot(h1, w2b_ref[...], preferred_element_type=jnp.float32)
                  + b2b_ref[...])


def _embed(coords_pad, fab, ws, wc, b1p, w2big, b2big):
    full = lambda shape: pl.BlockSpec(shape, lambda i: tuple(0 for _ in shape))
    return pl.pallas_call(
        _embed_body,
        grid=(_NPAD // _R_BLK,),
        in_specs=[
            pl.BlockSpec((_R_BLK, 2), lambda i: (i, 0)),
            full((2, 128)),
            full((128, _D)), full((128, _D)), full((1, _D)),
            full((_D, _D)), full((1, _D)),
        ],
        out_specs=pl.BlockSpec((_R_BLK, _D), lambda i: (i, 0)),
        out_shape=jax.ShapeDtypeStruct((_NPAD, _D), jnp.float32),
    )(coords_pad, fab, ws, wc, b1p, w2big, b2big)


def _pool_body(h_hbm, ids_hbm, bnd_hbm, pooled_hbm,
               acc, hv2, idsv2, idxv, bnd_v, hsem, isem):
    c = lax.axis_index("c")
    s = lax.axis_index("s")

    pltpu.sync_copy(bnd_hbm, bnd_v)
    bv = bnd_v[...]

    for p in range(2):
        u = c * 2 + p
        r0 = u * _R_SEG
        f0 = jnp.where(c == 0, bv[p], bv[2 + p])
        f1 = jnp.where(c == 0, bv[p + 1], bv[3 + p])
        a0 = (f0 // 8) * 8
        nblk = (f1 - a0 + (_CHUNK - 1)) // _CHUNK
        nmy = jnp.maximum((nblk - s + (_NS - 1)) // _NS, 0)

        for j in range(_CHUNK):
            hv2[0, j, :] = jnp.zeros((16,), jnp.float32)
        zoff = s * _ZSTRIPE
        for k in range(61):
            pltpu.sync_copy(hv2.at[0], acc.at[pl.ds(zoff + k * _CHUNK, _CHUNK)])
        pltpu.sync_copy(hv2.at[0, pl.ds(0, _ZTAIL)],
                        acc.at[pl.ds(zoff + 61 * _CHUNK, _ZTAIL)])
        plsc.subcore_barrier()

        def start(j, b):
            base = a0 + (s + j * _NS) * _CHUNK
            pltpu.async_copy(ids_hbm.at[pl.ds(base, _CHUNK)], idsv2.at[b],
                             isem.at[b])
            pltpu.async_copy(h_hbm.at[pl.ds(base, _CHUNK)], hv2.at[b],
                             hsem.at[b])

        def wait(b):
            pltpu.make_async_copy(ids_hbm.at[pl.ds(0, _CHUNK)], idsv2.at[b],
                                  isem.at[b]).wait()
            pltpu.make_async_copy(h_hbm.at[pl.ds(0, _CHUNK)], hv2.at[b],
                                  hsem.at[b]).wait()

        @pl.when(nmy > 0)
        def _():
            start(0, 0)

        def outer(k, carry):
            for b in range(2):
                j = 2 * k + b

                @pl.when(j < nmy)
                def _():
                    wait(b)

                    @pl.when(j + 1 < nmy)
                    def _():
                        start(j + 1, 1 - b)
                    for t in range(_CHUNK // 16):
                        iv = idsv2[b, pl.ds(t * 16, 16)]
                        ok = (iv >= r0) & (iv < r0 + _R_SEG)
                        idxv[pl.ds(t * 16, 16)] = jnp.where(ok, iv - r0, _DUMP)
                    pltpu.sync_copy(hv2.at[b], acc.at[idxv], add=True)
            return carry

        lax.fori_loop(0, (nmy + 1) // 2, outer, 0)
        plsc.subcore_barrier()

        doff = s * _DSTRIPE
        pltpu.sync_copy(acc.at[pl.ds(doff, _DSTRIPE)],
                        pooled_hbm.at[pl.ds(r0 + doff, _DSTRIPE)])
        plsc.subcore_barrier()


def _pool(h, ids_pad, bnd16):
    call = pl.kernel(
        _pool_body,
        out_type=jax.ShapeDtypeStruct((_S_PAD, _D), jnp.float32),
        mesh=plsc.VectorSubcoreMesh(core_axis_name="c", subcore_axis_name="s",
                                    num_cores=_NC, num_subcores=_NS),
        scratch_types=[
            pltpu.VMEM_SHARED((_SPM_ROWS, _D), jnp.float32),
            pltpu.VMEM((2, _CHUNK, _D), jnp.float32),
            pltpu.VMEM((2, _CHUNK), jnp.int32),
            pltpu.VMEM((_CHUNK,), jnp.int32),
            pltpu.VMEM((16,), jnp.int32),
            pltpu.SemaphoreType.DMA((2,)),
            pltpu.SemaphoreType.DMA((2,)),
        ],
        compiler_params=pltpu.CompilerParams(use_tc_tiling_on_sc=False),
    )
    return call(h, ids_pad, bnd16)


def _readout_body(p_ref, w3b_ref, b3t_ref, w4b_ref, b4t_ref, o_ref):
    pz = p_ref[...]
    zz = jax.nn.sigmoid(
        jnp.dot(pz, w3b_ref[...], preferred_element_type=jnp.float32)
        + b3t_ref[...])
    o_ref[...] = (jnp.dot(zz, w4b_ref[...], preferred_element_type=jnp.float32)
                  + b4t_ref[...])


def _readout(pooled, w3big, b3t, w4big, b4t):
    full = lambda shape: pl.BlockSpec(shape, lambda i: tuple(0 for _ in shape))
    rows = _S_PAD // _PK
    return pl.pallas_call(
        _readout_body,
        grid=(rows // _RO_BLK,),
        in_specs=[
            pl.BlockSpec((_RO_BLK, 128), lambda i: (i, 0)),
            full((128, 128)), full((1, 128)), full((128, _PK)),
            full((1, _PK)),
        ],
        out_specs=pl.BlockSpec((_RO_BLK, _PK), lambda i: (i, 0)),
        out_shape=jax.ShapeDtypeStruct((rows, _PK), jnp.float32),
    )(pooled, w3big, b3t, w4big, b4t)


def kernel(coordinates, local_cellxgene_ix, cell_n, gene_n, frequencies,
           shifts, W1, b1, W2, b2, W3, b3, W4, b4):
    f32 = jnp.float32
    n = coordinates.shape[0]
    ids = local_cellxgene_ix.astype(jnp.int32)

    coords_pad = jnp.zeros((_NPAD, 2), f32).at[:n].set(coordinates)
    ids_pad = jnp.full((_NPAD,), 1 << 30, jnp.int32).at[:n].set(ids)

    funiq = frequencies[0:100:2]
    fab = jnp.zeros((2, 128), f32)
    fab = fab.at[0, 0:50].set(funiq).at[1, 64:114].set(funiq)
    ws = jnp.zeros((128, _D), f32)
    ws = ws.at[0:50, 0:10].set(W1[:, 0:100:2].T)
    ws = ws.at[64:114, 0:10].set(W1[:, 100:200:2].T)
    wc = jnp.zeros((128, _D), f32)
    wc = wc.at[0:50, 0:10].set(W1[:, 1:100:2].T)
    wc = wc.at[64:114, 0:10].set(W1[:, 101:200:2].T)
    b1p = jnp.zeros((1, _D), f32).at[0, :10].set(b1)
    w2big = jnp.zeros((_D, _D), f32).at[:10, :10].set(W2.T)
    b2big = jnp.zeros((1, _D), f32).at[0, :10].set(b2)

    h = _embed(coords_pad, fab, ws, wc, b1p, w2big, b2big)

    bnds = jnp.searchsorted(
        ids, jnp.arange(0, _S_PAD + 1, _R_SEG, dtype=jnp.int32),
        side="left").astype(jnp.int32)
    bnd16 = jnp.zeros((16,), jnp.int32).at[:5].set(bnds)

    pooled = _pool(h, ids_pad, bnd16)

    w3t = jnp.zeros((_D, _D), f32).at[:10, :10].set(W3.T)
    w3big = jnp.kron(jnp.eye(_PK, dtype=f32), w3t)
    b3t = jnp.tile(jnp.zeros((1, _D), f32).at[0, :10].set(b3), (1, _PK))
    w4col = jnp.zeros((_D, 1), f32).at[:10, 0].set(W4[0])
    w4big = jnp.kron(jnp.eye(_PK, dtype=f32), w4col)
    b4t = jnp.full((1, _PK), b4[0], f32)

    out = _readout(pooled.reshape(_S_PAD // _PK, 128), w3big, b3t, w4big, b4t)
    return out.reshape(_S_PAD)[:_SEG_REAL].reshape(_CELL_N, _GENE_N)
